# baseline (speedup 1.0000x reference)
import functools

import jax
import jax.numpy as jnp
from jax.experimental import pallas as pl
from jax.experimental.pallas import tpu as pltpu

B = 64
OBS = 2048
TOTAL = 2304
PRED = TOTAL - OBS
A_DIM = 5
S_DIM = 2
Z_DIM = 16
K = 5
DELTA = 1e-4
H = 512
XFLAT = 192

ENC_TILE = 1024


def _enc_kernel(x_ref, w1_ref, b1_ref, wms_ref, bms_ref, out_ref):
    acc = b1_ref[...]
    for c in range(3):
        xc = x_ref[:, c, :] * 2.0 - 1.0
        acc = acc + jnp.dot(xc, w1_ref[pl.ds(64 * c, 64), :],
                            preferred_element_type=jnp.float32)
    h = jnp.tanh(acc)
    y = jnp.dot(h, wms_ref[...], preferred_element_type=jnp.float32) + bms_ref[...]
    col = jax.lax.broadcasted_iota(jnp.int32, y.shape, 1)
    out_ref[...] = jnp.where(col < Z_DIM, y, jnp.exp(y))


SCAN_CHUNK = 256


def _scan_kernel(act_ref, wst_ref, w1_ref, b1_ref, w2_ref, b2_ref,
                 stx_ref, sty_ref, st_scr):
    i = pl.program_id(0)

    @pl.when(i == 0)
    def _():
        st_scr[...] = jnp.zeros((B, S_DIM), jnp.float32)
        stx_ref[0:1, :] = jnp.zeros((1, B), jnp.float32)
        sty_ref[0:1, :] = jnp.zeros((1, B), jnp.float32)

    def step(t, st):
        rp = jnp.dot(act_ref[pl.ds(t * B, B), :], wst_ref[...],
                     preferred_element_type=jnp.float32)
        sp = st + rp
        hh = jnp.tanh(jnp.dot(sp, w1_ref[...],
                              preferred_element_type=jnp.float32) + b1_ref[...])
        g = jax.nn.sigmoid(jnp.dot(hh, w2_ref[...],
                                   preferred_element_type=jnp.float32)
                           + b2_ref[...])
        st_new = st + rp * g
        st_t = jnp.swapaxes(st_new, 0, 1)
        stx_ref[pl.ds(t, 1), :] = st_t[0:1]
        sty_ref[pl.ds(t, 1), :] = st_t[1:2]
        return st_new

    start = jnp.where(i == 0, 1, 0)
    st_scr[...] = jax.lax.fori_loop(start, SCAN_CHUNK, step, st_scr[...])


def _knn_kernel(q_ref, m_ref, zt_ref, ms_ref):
    q = q_ref[0]
    mt = m_ref[0]
    qx = q[:, 0:1]
    qy = q[:, 1:2]
    mx = mt[0:1, :]
    my = mt[1:2, :]
    dx = qx - mx
    dy = qy - my
    d = dx * dx + dy * dy

    u = jnp.zeros((PRED, OBS), jnp.float32)
    denom = jnp.zeros((PRED, 1), jnp.float32)
    for _ in range(K):
        mn = jnp.min(d, axis=1, keepdims=True)
        oneh = d <= mn
        wk = 1.0 / (mn + DELTA)
        u = u + jnp.where(oneh, wk, 0.0)
        denom = denom + wk
        d = jnp.where(oneh, jnp.float32(1e30), d)

    zt = zt_ref[0]
    numer = jnp.dot(u, zt, preferred_element_type=jnp.float32)
    ms_ref[0] = numer / denom


def _dec_kernel(ms_ref, eps_ref, wd1_ref, bd1_ref, wd2_ref, bd2_ref, out_ref):
    ms = jnp.transpose(ms_ref[...], (1, 0, 2))
    ms = ms.reshape(-1, 2 * Z_DIM)
    z = ms[:, :Z_DIM] + ms[:, Z_DIM:] * eps_ref[...]
    hd = jnp.tanh(jnp.dot(z, wd1_ref[...], preferred_element_type=jnp.float32)
                  + bd1_ref[...])
    xr = jnp.tanh(jnp.dot(hd, wd2_ref[...], preferred_element_type=jnp.float32)
                  + bd2_ref[...])
    out_ref[...] = (xr + 1.0) * 0.5


@jax.jit
def kernel(x_obs, actions, eps, W_enc1, b_enc1, W_mean, b_mean, W_std, b_std,
           W_st, W_sig1, b_sig1, W_sig2, b_sig2, W_dec1, b_dec1, W_dec2, b_dec2):
    f32 = jnp.float32
    nrows = B * OBS
    x_flat = x_obs.reshape(nrows, 3, 64)
    w_ms = jnp.concatenate([W_mean, W_std], axis=1)
    b_ms = jnp.concatenate([b_mean, b_std], axis=0).reshape(1, 2 * Z_DIM)
    b1r = b_enc1.reshape(1, H)

    zt = pl.pallas_call(
        _enc_kernel,
        grid=(nrows // ENC_TILE,),
        in_specs=[
            pl.BlockSpec((ENC_TILE, 3, 64), lambda i: (i, 0, 0)),
            pl.BlockSpec((XFLAT, H), lambda i: (0, 0)),
            pl.BlockSpec((1, H), lambda i: (0, 0)),
            pl.BlockSpec((H, 2 * Z_DIM), lambda i: (0, 0)),
            pl.BlockSpec((1, 2 * Z_DIM), lambda i: (0, 0)),
        ],
        out_specs=pl.BlockSpec((ENC_TILE, 2 * Z_DIM), lambda i: (i, 0)),
        out_shape=jax.ShapeDtypeStruct((nrows, 2 * Z_DIM), f32),
    )(x_flat, W_enc1, b1r, w_ms, b_ms)
    zt = zt.reshape(B, OBS, 2 * Z_DIM)

    act_r = actions.transpose(2, 0, 1).reshape(TOTAL * B, A_DIM)
    st_x, st_y = pl.pallas_call(
        _scan_kernel,
        grid=(TOTAL // SCAN_CHUNK,),
        in_specs=[
            pl.BlockSpec((SCAN_CHUNK * B, A_DIM), lambda i: (i, 0)),
            pl.BlockSpec((A_DIM, S_DIM), lambda i: (0, 0)),
            pl.BlockSpec((S_DIM, 10), lambda i: (0, 0)),
            pl.BlockSpec((1, 10), lambda i: (0, 0)),
            pl.BlockSpec((10, S_DIM), lambda i: (0, 0)),
            pl.BlockSpec((1, S_DIM), lambda i: (0, 0)),
        ],
        out_specs=[pl.BlockSpec((SCAN_CHUNK, B), lambda i: (i, 0)),
                   pl.BlockSpec((SCAN_CHUNK, B), lambda i: (i, 0))],
        out_shape=[jax.ShapeDtypeStruct((TOTAL, B), f32),
                   jax.ShapeDtypeStruct((TOTAL, B), f32)],
        scratch_shapes=[pltpu.VMEM((B, S_DIM), f32)],
    )(act_r, W_st, W_sig1, b_sig1.reshape(1, 10), W_sig2,
      b_sig2.reshape(1, S_DIM))

    q = jnp.stack([st_x[OBS:], st_y[OBS:]], axis=-1).transpose(1, 0, 2)
    m_t = jnp.stack([st_x[:OBS].T, st_y[:OBS].T], axis=1)

    ms = pl.pallas_call(
        _knn_kernel,
        grid=(B,),
        in_specs=[
            pl.BlockSpec((1, PRED, S_DIM), lambda b: (b, 0, 0)),
            pl.BlockSpec((1, S_DIM, OBS), lambda b: (b, 0, 0)),
            pl.BlockSpec((1, OBS, 2 * Z_DIM), lambda b: (b, 0, 0)),
        ],
        out_specs=pl.BlockSpec((1, PRED, 2 * Z_DIM), lambda b: (b, 0, 0)),
        out_shape=jax.ShapeDtypeStruct((B, PRED, 2 * Z_DIM), f32),
    )(q, m_t, zt)

    PT = 64
    out = pl.pallas_call(
        _dec_kernel,
        grid=(PRED // PT,),
        in_specs=[
            pl.BlockSpec((B, PT, 2 * Z_DIM), lambda p: (0, p, 0)),
            pl.BlockSpec((PT * B, Z_DIM), lambda p: (p, 0)),
            pl.BlockSpec((Z_DIM, H), lambda p: (0, 0)),
            pl.BlockSpec((1, H), lambda p: (0, 0)),
            pl.BlockSpec((H, XFLAT), lambda p: (0, 0)),
            pl.BlockSpec((1, XFLAT), lambda p: (0, 0)),
        ],
        out_specs=pl.BlockSpec((PT * B, XFLAT), lambda p: (p, 0)),
        out_shape=jax.ShapeDtypeStruct((PRED * B, XFLAT), f32),
    )(ms, eps.reshape(PRED * B, Z_DIM), W_dec1, b_dec1.reshape(1, H),
      W_dec2, b_dec2.reshape(1, XFLAT))

    return out.reshape(PRED, B, 3, 8, 8)


# speedup vs baseline: 1.1444x; 1.1444x over previous
import functools

import jax
import jax.numpy as jnp
from jax.experimental import pallas as pl
from jax.experimental.pallas import tpu as pltpu

B = 64
OBS = 2048
TOTAL = 2304
PRED = TOTAL - OBS
A_DIM = 5
S_DIM = 2
Z_DIM = 16
K = 5
DELTA = 1e-4
H = 512
XFLAT = 192

ENC_TILE = 1024


def _enc_kernel(x_ref, w1_ref, b1_ref, wms_ref, bms_ref, out_ref):
    acc = b1_ref[...]
    for c in range(3):
        xc = x_ref[:, c, :] * 2.0 - 1.0
        acc = acc + jnp.dot(xc, w1_ref[pl.ds(64 * c, 64), :],
                            preferred_element_type=jnp.float32)
    h = jnp.tanh(acc)
    y = jnp.dot(h, wms_ref[...], preferred_element_type=jnp.float32) + bms_ref[...]
    col = jax.lax.broadcasted_iota(jnp.int32, y.shape, 1)
    out_ref[...] = jnp.where(col < Z_DIM, y, jnp.exp(y))


SCAN_CHUNK = 256


def _scan_kernel(act_ref, wst_ref, w1_ref, b1_ref, w2_ref, b2_ref,
                 stx_ref, sty_ref, st_scr, repl_ref):
    i = pl.program_id(0)

    @pl.when(i == 0)
    def _():
        st_scr[...] = jnp.zeros((B, S_DIM), jnp.float32)
        stx_ref[0:1, :] = jnp.zeros((1, B), jnp.float32)
        sty_ref[0:1, :] = jnp.zeros((1, B), jnp.float32)

    repl_ref[...] = jnp.dot(act_ref[...], wst_ref[...],
                            preferred_element_type=jnp.float32)
    w1 = w1_ref[...]
    b1 = b1_ref[...]
    w2 = w2_ref[...]
    b2 = b2_ref[...]

    def step(t, st):
        rp = repl_ref[pl.ds(t * B, B), :]
        sp = st + rp
        hh = jnp.tanh(jnp.dot(sp, w1, preferred_element_type=jnp.float32)
                      + b1)
        g = jax.nn.sigmoid(jnp.dot(hh, w2, preferred_element_type=jnp.float32)
                           + b2)
        st_new = st + rp * g
        st_t = jnp.swapaxes(st_new, 0, 1)
        stx_ref[pl.ds(t, 1), :] = st_t[0:1]
        sty_ref[pl.ds(t, 1), :] = st_t[1:2]
        return st_new

    start = jnp.where(i == 0, 1, 0)
    st_scr[...] = jax.lax.fori_loop(start, SCAN_CHUNK, step, st_scr[...])


def _knn_kernel(q_ref, m_ref, zt_ref, ms_ref):
    q = q_ref[0]
    mt = m_ref[0]
    qx = q[:, 0:1]
    qy = q[:, 1:2]
    mx = mt[0:1, :]
    my = mt[1:2, :]
    dx = qx - mx
    dy = qy - my
    d = dx * dx + dy * dy

    u = jnp.zeros((PRED, OBS), jnp.float32)
    denom = jnp.zeros((PRED, 1), jnp.float32)
    for _ in range(K):
        mn = jnp.min(d, axis=1, keepdims=True)
        oneh = d <= mn
        wk = 1.0 / (mn + DELTA)
        u = u + jnp.where(oneh, wk, 0.0)
        denom = denom + wk
        d = jnp.where(oneh, jnp.float32(1e30), d)

    zt = zt_ref[0]
    numer = jnp.dot(u, zt, preferred_element_type=jnp.float32)
    ms_ref[0] = numer / denom


def _dec_kernel(ms_ref, eps_ref, wd1_ref, bd1_ref, wd2_ref, bd2_ref, out_ref):
    ms = jnp.transpose(ms_ref[...], (1, 0, 2))
    ms = ms.reshape(-1, 2 * Z_DIM)
    z = ms[:, :Z_DIM] + ms[:, Z_DIM:] * eps_ref[...]
    hd = jnp.tanh(jnp.dot(z, wd1_ref[...], preferred_element_type=jnp.float32)
                  + bd1_ref[...])
    xr = jnp.tanh(jnp.dot(hd, wd2_ref[...], preferred_element_type=jnp.float32)
                  + bd2_ref[...])
    out_ref[...] = (xr + 1.0) * 0.5


@jax.jit
def kernel(x_obs, actions, eps, W_enc1, b_enc1, W_mean, b_mean, W_std, b_std,
           W_st, W_sig1, b_sig1, W_sig2, b_sig2, W_dec1, b_dec1, W_dec2, b_dec2):
    f32 = jnp.float32
    nrows = B * OBS
    x_flat = x_obs.reshape(nrows, 3, 64)
    w_ms = jnp.concatenate([W_mean, W_std], axis=1)
    b_ms = jnp.concatenate([b_mean, b_std], axis=0).reshape(1, 2 * Z_DIM)
    b1r = b_enc1.reshape(1, H)

    zt = pl.pallas_call(
        _enc_kernel,
        grid=(nrows // ENC_TILE,),
        in_specs=[
            pl.BlockSpec((ENC_TILE, 3, 64), lambda i: (i, 0, 0)),
            pl.BlockSpec((XFLAT, H), lambda i: (0, 0)),
            pl.BlockSpec((1, H), lambda i: (0, 0)),
            pl.BlockSpec((H, 2 * Z_DIM), lambda i: (0, 0)),
            pl.BlockSpec((1, 2 * Z_DIM), lambda i: (0, 0)),
        ],
        out_specs=pl.BlockSpec((ENC_TILE, 2 * Z_DIM), lambda i: (i, 0)),
        out_shape=jax.ShapeDtypeStruct((nrows, 2 * Z_DIM), f32),
    )(x_flat, W_enc1, b1r, w_ms, b_ms)
    zt = zt.reshape(B, OBS, 2 * Z_DIM)

    act_r = actions.transpose(2, 0, 1).reshape(TOTAL * B, A_DIM)
    st_x, st_y = pl.pallas_call(
        _scan_kernel,
        grid=(TOTAL // SCAN_CHUNK,),
        in_specs=[
            pl.BlockSpec((SCAN_CHUNK * B, A_DIM), lambda i: (i, 0)),
            pl.BlockSpec((A_DIM, S_DIM), lambda i: (0, 0)),
            pl.BlockSpec((S_DIM, 10), lambda i: (0, 0)),
            pl.BlockSpec((1, 10), lambda i: (0, 0)),
            pl.BlockSpec((10, S_DIM), lambda i: (0, 0)),
            pl.BlockSpec((1, S_DIM), lambda i: (0, 0)),
        ],
        out_specs=[pl.BlockSpec((SCAN_CHUNK, B), lambda i: (i, 0)),
                   pl.BlockSpec((SCAN_CHUNK, B), lambda i: (i, 0))],
        out_shape=[jax.ShapeDtypeStruct((TOTAL, B), f32),
                   jax.ShapeDtypeStruct((TOTAL, B), f32)],
        scratch_shapes=[pltpu.VMEM((B, S_DIM), f32),
                        pltpu.VMEM((SCAN_CHUNK * B, S_DIM), f32)],
    )(act_r, W_st, W_sig1, b_sig1.reshape(1, 10), W_sig2,
      b_sig2.reshape(1, S_DIM))

    q = jnp.stack([st_x[OBS:], st_y[OBS:]], axis=-1).transpose(1, 0, 2)
    m_t = jnp.stack([st_x[:OBS].T, st_y[:OBS].T], axis=1)

    ms = pl.pallas_call(
        _knn_kernel,
        grid=(B,),
        in_specs=[
            pl.BlockSpec((1, PRED, S_DIM), lambda b: (b, 0, 0)),
            pl.BlockSpec((1, S_DIM, OBS), lambda b: (b, 0, 0)),
            pl.BlockSpec((1, OBS, 2 * Z_DIM), lambda b: (b, 0, 0)),
        ],
        out_specs=pl.BlockSpec((1, PRED, 2 * Z_DIM), lambda b: (b, 0, 0)),
        out_shape=jax.ShapeDtypeStruct((B, PRED, 2 * Z_DIM), f32),
    )(q, m_t, zt)

    PT = 64
    out = pl.pallas_call(
        _dec_kernel,
        grid=(PRED // PT,),
        in_specs=[
            pl.BlockSpec((B, PT, 2 * Z_DIM), lambda p: (0, p, 0)),
            pl.BlockSpec((PT * B, Z_DIM), lambda p: (p, 0)),
            pl.BlockSpec((Z_DIM, H), lambda p: (0, 0)),
            pl.BlockSpec((1, H), lambda p: (0, 0)),
            pl.BlockSpec((H, XFLAT), lambda p: (0, 0)),
            pl.BlockSpec((1, XFLAT), lambda p: (0, 0)),
        ],
        out_specs=pl.BlockSpec((PT * B, XFLAT), lambda p: (p, 0)),
        out_shape=jax.ShapeDtypeStruct((PRED * B, XFLAT), f32),
    )(ms, eps.reshape(PRED * B, Z_DIM), W_dec1, b_dec1.reshape(1, H),
      W_dec2, b_dec2.reshape(1, XFLAT))

    return out.reshape(PRED, B, 3, 8, 8)


# speedup vs baseline: 1.9825x; 1.7323x over previous
import functools

import jax
import jax.numpy as jnp
from jax.experimental import pallas as pl
from jax.experimental.pallas import tpu as pltpu

B = 64
OBS = 2048
TOTAL = 2304
PRED = TOTAL - OBS
A_DIM = 5
S_DIM = 2
Z_DIM = 16
K = 5
DELTA = 1e-4
H = 512
XFLAT = 192

ENC_TILE = 1024


def _enc_kernel(x_ref, w1_ref, b1_ref, wms_ref, bms_ref, out_ref):
    acc = b1_ref[...]
    for c in range(3):
        xc = x_ref[:, c, :] * 2.0 - 1.0
        acc = acc + jnp.dot(xc, w1_ref[pl.ds(64 * c, 64), :],
                            preferred_element_type=jnp.float32)
    h = jnp.tanh(acc)
    y = jnp.dot(h, wms_ref[...], preferred_element_type=jnp.float32) + bms_ref[...]
    col = jax.lax.broadcasted_iota(jnp.int32, y.shape, 1)
    out_ref[...] = jnp.where(col < Z_DIM, y, jnp.exp(y))


def _bq(a):
    return a.astype(jnp.bfloat16).astype(jnp.float32)


def _scan_kernel(act_ref, wst_ref, w1_ref, b1_ref, w2_ref, b2_ref,
                 stx_ref, sty_ref, rx_ref, ry_ref):
    rx = jnp.zeros((TOTAL, B), jnp.float32)
    ry = jnp.zeros((TOTAL, B), jnp.float32)
    for a in range(A_DIM):
        act_a = _bq(act_ref[a])
        rx = rx + act_a * wst_ref[a, 0]
        ry = ry + act_a * wst_ref[a, 1]
    rx_ref[...] = rx
    ry_ref[...] = ry
    stx_ref[0:1, :] = jnp.zeros((1, B), jnp.float32)
    sty_ref[0:1, :] = jnp.zeros((1, B), jnp.float32)

    def step(t, st):
        stx, sty = st
        rpx = rx_ref[pl.ds(t, 1), :]
        rpy = ry_ref[pl.ds(t, 1), :]
        spx = stx + rpx
        spy = sty + rpy
        sqx = _bq(spx)
        sqy = _bq(spy)
        px = []
        py = []
        for k in range(10):
            hk = jnp.tanh(sqx * w1_ref[0, k] + sqy * w1_ref[1, k]
                          + b1_ref[0, k])
            hq = _bq(hk)
            px.append(hq * w2_ref[k, 0])
            py.append(hq * w2_ref[k, 1])

        def tree(ps):
            while len(ps) > 1:
                nxt = [ps[i] + ps[i + 1] for i in range(0, len(ps) - 1, 2)]
                if len(ps) % 2:
                    nxt.append(ps[-1])
                ps = nxt
            return ps[0]

        stx = stx + rpx * jax.nn.sigmoid(tree(px) + b2_ref[0, 0])
        sty = sty + rpy * jax.nn.sigmoid(tree(py) + b2_ref[0, 1])
        stx_ref[pl.ds(t, 1), :] = stx
        sty_ref[pl.ds(t, 1), :] = sty
        return stx, sty

    jax.lax.fori_loop(1, TOTAL, step,
                      (jnp.zeros((1, B), jnp.float32),
                       jnp.zeros((1, B), jnp.float32)))


def _knn_kernel(q_ref, m_ref, zt_ref, ms_ref):
    q = q_ref[0]
    mt = m_ref[0]
    qx = q[:, 0:1]
    qy = q[:, 1:2]
    mx = mt[0:1, :]
    my = mt[1:2, :]
    dx = qx - mx
    dy = qy - my
    d = dx * dx + dy * dy

    u = jnp.zeros((PRED, OBS), jnp.float32)
    denom = jnp.zeros((PRED, 1), jnp.float32)
    for _ in range(K):
        mn = jnp.min(d, axis=1, keepdims=True)
        oneh = d <= mn
        wk = 1.0 / (mn + DELTA)
        u = u + jnp.where(oneh, wk, 0.0)
        denom = denom + wk
        d = jnp.where(oneh, jnp.float32(1e30), d)

    zt = zt_ref[0]
    numer = jnp.dot(u, zt, preferred_element_type=jnp.float32)
    ms_ref[0] = numer / denom


def _dec_kernel(ms_ref, eps_ref, wd1_ref, bd1_ref, wd2_ref, bd2_ref, out_ref):
    ms = jnp.transpose(ms_ref[...], (1, 0, 2))
    ms = ms.reshape(-1, 2 * Z_DIM)
    z = ms[:, :Z_DIM] + ms[:, Z_DIM:] * eps_ref[...]
    hd = jnp.tanh(jnp.dot(z, wd1_ref[...], preferred_element_type=jnp.float32)
                  + bd1_ref[...])
    xr = jnp.tanh(jnp.dot(hd, wd2_ref[...], preferred_element_type=jnp.float32)
                  + bd2_ref[...])
    out_ref[...] = (xr + 1.0) * 0.5


@jax.jit
def kernel(x_obs, actions, eps, W_enc1, b_enc1, W_mean, b_mean, W_std, b_std,
           W_st, W_sig1, b_sig1, W_sig2, b_sig2, W_dec1, b_dec1, W_dec2, b_dec2):
    f32 = jnp.float32
    nrows = B * OBS
    x_flat = x_obs.reshape(nrows, 3, 64)
    w_ms = jnp.concatenate([W_mean, W_std], axis=1)
    b_ms = jnp.concatenate([b_mean, b_std], axis=0).reshape(1, 2 * Z_DIM)
    b1r = b_enc1.reshape(1, H)

    zt = pl.pallas_call(
        _enc_kernel,
        grid=(nrows // ENC_TILE,),
        in_specs=[
            pl.BlockSpec((ENC_TILE, 3, 64), lambda i: (i, 0, 0)),
            pl.BlockSpec((XFLAT, H), lambda i: (0, 0)),
            pl.BlockSpec((1, H), lambda i: (0, 0)),
            pl.BlockSpec((H, 2 * Z_DIM), lambda i: (0, 0)),
            pl.BlockSpec((1, 2 * Z_DIM), lambda i: (0, 0)),
        ],
        out_specs=pl.BlockSpec((ENC_TILE, 2 * Z_DIM), lambda i: (i, 0)),
        out_shape=jax.ShapeDtypeStruct((nrows, 2 * Z_DIM), f32),
    )(x_flat, W_enc1, b1r, w_ms, b_ms)
    zt = zt.reshape(B, OBS, 2 * Z_DIM)

    bfq = lambda a: a.astype(jnp.bfloat16).astype(f32)
    act_r = actions.transpose(1, 2, 0)
    smem_spec = pl.BlockSpec(memory_space=pltpu.SMEM)
    st_x, st_y = pl.pallas_call(
        _scan_kernel,
        grid=(1,),
        in_specs=[
            pl.BlockSpec((A_DIM, TOTAL, B), lambda i: (0, 0, 0)),
            smem_spec, smem_spec, smem_spec, smem_spec, smem_spec,
        ],
        out_specs=[pl.BlockSpec((TOTAL, B), lambda i: (0, 0)),
                   pl.BlockSpec((TOTAL, B), lambda i: (0, 0))],
        out_shape=[jax.ShapeDtypeStruct((TOTAL, B), f32),
                   jax.ShapeDtypeStruct((TOTAL, B), f32)],
        scratch_shapes=[pltpu.VMEM((TOTAL, B), f32),
                        pltpu.VMEM((TOTAL, B), f32)],
    )(act_r, bfq(W_st), bfq(W_sig1), b_sig1.reshape(1, 10), bfq(W_sig2),
      b_sig2.reshape(1, S_DIM))

    q = jnp.stack([st_x[OBS:], st_y[OBS:]], axis=-1).transpose(1, 0, 2)
    m_t = jnp.stack([st_x[:OBS].T, st_y[:OBS].T], axis=1)

    ms = pl.pallas_call(
        _knn_kernel,
        grid=(B,),
        in_specs=[
            pl.BlockSpec((1, PRED, S_DIM), lambda b: (b, 0, 0)),
            pl.BlockSpec((1, S_DIM, OBS), lambda b: (b, 0, 0)),
            pl.BlockSpec((1, OBS, 2 * Z_DIM), lambda b: (b, 0, 0)),
        ],
        out_specs=pl.BlockSpec((1, PRED, 2 * Z_DIM), lambda b: (b, 0, 0)),
        out_shape=jax.ShapeDtypeStruct((B, PRED, 2 * Z_DIM), f32),
    )(q, m_t, zt)

    PT = 64
    out = pl.pallas_call(
        _dec_kernel,
        grid=(PRED // PT,),
        in_specs=[
            pl.BlockSpec((B, PT, 2 * Z_DIM), lambda p: (0, p, 0)),
            pl.BlockSpec((PT * B, Z_DIM), lambda p: (p, 0)),
            pl.BlockSpec((Z_DIM, H), lambda p: (0, 0)),
            pl.BlockSpec((1, H), lambda p: (0, 0)),
            pl.BlockSpec((H, XFLAT), lambda p: (0, 0)),
            pl.BlockSpec((1, XFLAT), lambda p: (0, 0)),
        ],
        out_specs=pl.BlockSpec((PT * B, XFLAT), lambda p: (p, 0)),
        out_shape=jax.ShapeDtypeStruct((PRED * B, XFLAT), f32),
    )(ms, eps.reshape(PRED * B, Z_DIM), W_dec1, b_dec1.reshape(1, H),
      W_dec2, b_dec2.reshape(1, XFLAT))

    return out.reshape(PRED, B, 3, 8, 8)


# speedup vs baseline: 3.0283x; 1.5275x over previous
import functools

import jax
import jax.numpy as jnp
from jax.experimental import pallas as pl
from jax.experimental.pallas import tpu as pltpu

B = 64
OBS = 2048
TOTAL = 2304
PRED = TOTAL - OBS
A_DIM = 5
S_DIM = 2
Z_DIM = 16
K = 5
DELTA = 1e-4
H = 512
XFLAT = 192

ENC_TILE = 1024


def _enc_kernel(x_ref, w1_ref, b1_ref, wms_ref, bms_ref, out_ref):
    x = x_ref[...] * 2.0 - 1.0
    h = jnp.tanh(jnp.dot(x, w1_ref[...], preferred_element_type=jnp.float32)
                 + b1_ref[...])
    y = jnp.dot(h, wms_ref[...], preferred_element_type=jnp.float32) + bms_ref[...]
    col = jax.lax.broadcasted_iota(jnp.int32, y.shape, 1)
    out_ref[...] = jnp.where(col < Z_DIM, y, jnp.exp(y))


def _bq(a):
    return a.astype(jnp.bfloat16).astype(jnp.float32)


def _scan_kernel(act_ref, wst_ref, w1_ref, b1_ref, w2_ref, b2_ref,
                 stx_ref, sty_ref, rx_ref, ry_ref):
    rx = jnp.zeros((TOTAL, B), jnp.float32)
    ry = jnp.zeros((TOTAL, B), jnp.float32)
    for a in range(A_DIM):
        act_a = _bq(act_ref[a])
        rx = rx + act_a * wst_ref[a, 0]
        ry = ry + act_a * wst_ref[a, 1]
    rx_ref[...] = rx
    ry_ref[...] = ry
    stx_ref[0:1, :] = jnp.zeros((1, B), jnp.float32)
    sty_ref[0:1, :] = jnp.zeros((1, B), jnp.float32)

    def step(t, st):
        stx, sty = st
        rpx = rx_ref[pl.ds(t, 1), :]
        rpy = ry_ref[pl.ds(t, 1), :]
        spx = stx + rpx
        spy = sty + rpy
        sqx = _bq(spx)
        sqy = _bq(spy)
        px = []
        py = []
        for k in range(10):
            hk = jnp.tanh(sqx * w1_ref[0, k] + sqy * w1_ref[1, k]
                          + b1_ref[0, k])
            hq = _bq(hk)
            px.append(hq * w2_ref[k, 0])
            py.append(hq * w2_ref[k, 1])

        def tree(ps):
            while len(ps) > 1:
                nxt = [ps[i] + ps[i + 1] for i in range(0, len(ps) - 1, 2)]
                if len(ps) % 2:
                    nxt.append(ps[-1])
                ps = nxt
            return ps[0]

        stx = stx + rpx * jax.nn.sigmoid(tree(px) + b2_ref[0, 0])
        sty = sty + rpy * jax.nn.sigmoid(tree(py) + b2_ref[0, 1])
        stx_ref[pl.ds(t, 1), :] = stx
        sty_ref[pl.ds(t, 1), :] = sty
        return stx, sty

    jax.lax.fori_loop(1, TOTAL, step,
                      (jnp.zeros((1, B), jnp.float32),
                       jnp.zeros((1, B), jnp.float32)))


def _knn_kernel(q_ref, m_ref, zt_ref, ms_ref):
    q = q_ref[0]
    mt = m_ref[0]
    qx = q[:, 0:1]
    qy = q[:, 1:2]
    mx = mt[0:1, :]
    my = mt[1:2, :]
    dx = qx - mx
    dy = qy - my
    d = dx * dx + dy * dy

    u = jnp.zeros((PRED, OBS), jnp.float32)
    denom = jnp.zeros((PRED, 1), jnp.float32)
    for _ in range(K):
        mn = jnp.min(d, axis=1, keepdims=True)
        oneh = d <= mn
        wk = 1.0 / (mn + DELTA)
        u = u + jnp.where(oneh, wk, 0.0)
        denom = denom + wk
        d = jnp.where(oneh, jnp.float32(1e30), d)

    zt = zt_ref[0]
    numer = jnp.dot(u, zt, preferred_element_type=jnp.float32)
    ms_ref[0] = numer / denom


def _dec_kernel(ms_ref, eps_ref, wd1_ref, bd1_ref, wd2_ref, bd2_ref, out_ref):
    ms = jnp.transpose(ms_ref[...], (1, 0, 2))
    ms = ms.reshape(-1, 2 * Z_DIM)
    z = ms[:, :Z_DIM] + ms[:, Z_DIM:] * eps_ref[...]
    hd = jnp.tanh(jnp.dot(z, wd1_ref[...], preferred_element_type=jnp.float32)
                  + bd1_ref[...])
    xr = jnp.tanh(jnp.dot(hd, wd2_ref[...], preferred_element_type=jnp.float32)
                  + bd2_ref[...])
    out_ref[...] = (xr + 1.0) * 0.5


@jax.jit
def kernel(x_obs, actions, eps, W_enc1, b_enc1, W_mean, b_mean, W_std, b_std,
           W_st, W_sig1, b_sig1, W_sig2, b_sig2, W_dec1, b_dec1, W_dec2, b_dec2):
    f32 = jnp.float32
    nrows = B * OBS
    x_flat = x_obs.reshape(nrows, XFLAT)
    w_ms = jnp.concatenate([W_mean, W_std], axis=1)
    b_ms = jnp.concatenate([b_mean, b_std], axis=0).reshape(1, 2 * Z_DIM)
    b1r = b_enc1.reshape(1, H)

    zt = pl.pallas_call(
        _enc_kernel,
        grid=(nrows // ENC_TILE,),
        in_specs=[
            pl.BlockSpec((ENC_TILE, XFLAT), lambda i: (i, 0)),
            pl.BlockSpec((XFLAT, H), lambda i: (0, 0)),
            pl.BlockSpec((1, H), lambda i: (0, 0)),
            pl.BlockSpec((H, 2 * Z_DIM), lambda i: (0, 0)),
            pl.BlockSpec((1, 2 * Z_DIM), lambda i: (0, 0)),
        ],
        out_specs=pl.BlockSpec((ENC_TILE, 2 * Z_DIM), lambda i: (i, 0)),
        out_shape=jax.ShapeDtypeStruct((nrows, 2 * Z_DIM), f32),
    )(x_flat, W_enc1, b1r, w_ms, b_ms)
    zt = zt.reshape(B, OBS, 2 * Z_DIM)

    bfq = lambda a: a.astype(jnp.bfloat16).astype(f32)
    act_r = actions.transpose(1, 2, 0)
    smem_spec = pl.BlockSpec(memory_space=pltpu.SMEM)
    st_x, st_y = pl.pallas_call(
        _scan_kernel,
        grid=(1,),
        in_specs=[
            pl.BlockSpec((A_DIM, TOTAL, B), lambda i: (0, 0, 0)),
            smem_spec, smem_spec, smem_spec, smem_spec, smem_spec,
        ],
        out_specs=[pl.BlockSpec((TOTAL, B), lambda i: (0, 0)),
                   pl.BlockSpec((TOTAL, B), lambda i: (0, 0))],
        out_shape=[jax.ShapeDtypeStruct((TOTAL, B), f32),
                   jax.ShapeDtypeStruct((TOTAL, B), f32)],
        scratch_shapes=[pltpu.VMEM((TOTAL, B), f32),
                        pltpu.VMEM((TOTAL, B), f32)],
    )(act_r, bfq(W_st), bfq(W_sig1), b_sig1.reshape(1, 10), bfq(W_sig2),
      b_sig2.reshape(1, S_DIM))

    q = jnp.stack([st_x[OBS:], st_y[OBS:]], axis=-1).transpose(1, 0, 2)
    m_t = jnp.stack([st_x[:OBS].T, st_y[:OBS].T], axis=1)

    ms = pl.pallas_call(
        _knn_kernel,
        grid=(B,),
        in_specs=[
            pl.BlockSpec((1, PRED, S_DIM), lambda b: (b, 0, 0)),
            pl.BlockSpec((1, S_DIM, OBS), lambda b: (b, 0, 0)),
            pl.BlockSpec((1, OBS, 2 * Z_DIM), lambda b: (b, 0, 0)),
        ],
        out_specs=pl.BlockSpec((1, PRED, 2 * Z_DIM), lambda b: (b, 0, 0)),
        out_shape=jax.ShapeDtypeStruct((B, PRED, 2 * Z_DIM), f32),
    )(q, m_t, zt)

    PT = 64
    out = pl.pallas_call(
        _dec_kernel,
        grid=(PRED // PT,),
        in_specs=[
            pl.BlockSpec((B, PT, 2 * Z_DIM), lambda p: (0, p, 0)),
            pl.BlockSpec((PT * B, Z_DIM), lambda p: (p, 0)),
            pl.BlockSpec((Z_DIM, H), lambda p: (0, 0)),
            pl.BlockSpec((1, H), lambda p: (0, 0)),
            pl.BlockSpec((H, XFLAT), lambda p: (0, 0)),
            pl.BlockSpec((1, XFLAT), lambda p: (0, 0)),
        ],
        out_specs=pl.BlockSpec((PT * B, XFLAT), lambda p: (p, 0)),
        out_shape=jax.ShapeDtypeStruct((PRED * B, XFLAT), f32),
    )(ms, eps.reshape(PRED * B, Z_DIM), W_dec1, b_dec1.reshape(1, H),
      W_dec2, b_dec2.reshape(1, XFLAT))

    return out.reshape(PRED, B, 3, 8, 8)
